# baseline (speedup 1.0000x reference)
import functools

import jax
import jax.numpy as jnp
from jax import lax
from jax.experimental import pallas as pl
from jax.experimental.pallas import tpu as pltpu
from jax.experimental.pallas import tpu_sc as plsc

N_VOCAB = 1000000
N_EMBED = 32
BATCH = 16384

_INFO = plsc.get_sparse_core_info()
_NC = _INFO.num_cores
_NS = _INFO.num_subcores
_NW = _NC * _NS
_BPW = BATCH // _NW

_NBLK = (N_VOCAB + 127) // 128
_BLK_BASE = _NBLK // _NW
_BLK_EXTRA = _NBLK - _BLK_BASE * _NW
_TAIL_LO = (N_VOCAB // 128) * 128
_CH = 1024
_NCHUNK = 31

_CAP = 1024
_NSELW = _CAP // 16
_HITROWS = _CAP * N_EMBED // 128
_SELROWS = 16
_TBLK = _HITROWS + _SELROWS
_WBLK = 2 * _TBLK
_STAGE_ROWS = _NW * _WBLK

_IOTA = lambda: lax.iota(jnp.int32, 16)


def _splat(x):
    return jnp.full((16,), x, jnp.int32)


def _chunk_select(sel_idx, chl, cnt, start, width):
    iota = _IOTA()

    def win(j, chcnt):
        sv = sel_idx[pl.ds(j * 16, 16)]
        pos = j * 16 + iota
        m = (pos < cnt) & (sv >= start) & (sv < start + width)
        e2 = pos * 1024 + (sv - start)
        lane = plsc.cumsum(m.astype(jnp.int32)) - 1
        plsc.store_scatter(
            chl, [jnp.where(m, chcnt + lane, 0)], e2, mask=m
        )
        return chcnt + plsc.all_reduce_population_count(m)[0]

    return lax.fori_loop(0, _NSELW, win, jnp.int32(0))


def _scan_chunk(sel_idx, chl, block_v, hitrows_v, cnt, start):
    iota = _IOTA()
    chcnt = _chunk_select(sel_idx, chl, cnt, start, _CH)

    def extract(j2, carry):
        ev = chl[pl.ds(j2 * 16, 16)]
        m2 = (j2 * 16 + iota) < chcnt
        iloc = jnp.where(m2, ev & 1023, 0)
        slot = ev >> 10
        for c in range(N_EMBED):
            fp = slot * N_EMBED + c
            vals = plsc.load_gather(block_v, [_splat(c), iloc], mask=m2)
            plsc.store_scatter(
                hitrows_v, [fp >> 7, fp & 127], vals, mask=m2
            )
        return carry

    lax.fori_loop(0, 4, extract, jnp.int32(0))


def _scan_tail(sel_idx, chl, tailv, hitrows_v, cnt):
    iota = _IOTA()
    chcnt = _chunk_select(sel_idx, chl, cnt, jnp.int32(_TAIL_LO),
                          N_VOCAB - _TAIL_LO)

    def extract(j2, carry):
        ev = chl[pl.ds(j2 * 16, 16)]
        m2 = (j2 * 16 + iota) < chcnt
        iloc = jnp.where(m2, ev & 1023, 0)
        slot = ev >> 10
        for c in range(N_EMBED):
            fps = iloc * N_EMBED + c
            fp = slot * N_EMBED + c
            vals = plsc.load_gather(tailv, [fps >> 7, fps & 127], mask=m2)
            plsc.store_scatter(
                hitrows_v, [fp >> 7, fp & 127], vals, mask=m2
            )
        return carry

    lax.fori_loop(0, 4, extract, jnp.int32(0))


def _k1_body(idx0_hbm, idx1_hbm, tab0_hbm, tab1_hbm, tail0_hbm, tail1_hbm,
             stage_hbm,
             idxv, sel_idx, selb2d, chl, block_v, hitrows_v, tailv):
    wid = lax.axis_index("s") * _NC + lax.axis_index("c")
    iota = _IOTA()

    start_block = wid * _BLK_BASE + lax.min(wid, _BLK_EXTRA)
    nblocks = _BLK_BASE + (wid < _BLK_EXTRA).astype(jnp.int32)
    lo = start_block * 128
    fetch_hi = lax.min(lo + nblocks * 128, jnp.int32(_TAIL_LO))
    hi_sel = lax.min(lo + nblocks * 128, jnp.int32(N_VOCAB))

    for t, (idx_hbm, tab_hbm, tail_hbm) in enumerate(
        ((idx0_hbm, tab0_hbm, tail0_hbm), (idx1_hbm, tab1_hbm, tail1_hbm))
    ):
        pltpu.sync_copy(idx_hbm, idxv)

        def sel_step(k, cnt):
            v = idxv[pl.ds(k * 16, 16)]
            m = (v >= lo) & (v < hi_sel)
            lane = plsc.cumsum(m.astype(jnp.int32)) - 1
            tgt = jnp.where(m, cnt + lane, 0)
            plsc.store_scatter(sel_idx, [tgt], v, mask=m)
            plsc.store_scatter(
                selb2d,
                [tgt >> 7, tgt & 127],
                plsc.bitcast(k * 16 + iota, jnp.float32),
                mask=m,
            )
            return cnt + plsc.all_reduce_population_count(m)[0]

        cnt = lax.fori_loop(0, BATCH // 16, sel_step, jnp.int32(0))

        selb2d[8, pl.ds(0, 16)] = plsc.bitcast(_splat(cnt), jnp.float32)
        rowbase = wid * _WBLK + t * _TBLK
        pltpu.sync_copy(selb2d, stage_hbm.at[pl.ds(rowbase + _HITROWS,
                                                   _SELROWS)])

        def chunk_step(ci, carry):
            start = lax.min(lo + ci * _CH, fetch_hi - _CH)
            start = pl.multiple_of(start, 128)
            pltpu.sync_copy(tab_hbm.at[:, pl.ds(start, _CH)], block_v)
            _scan_chunk(sel_idx, chl, block_v, hitrows_v, cnt, start)
            return carry

        lax.fori_loop(0, _NCHUNK, chunk_step, jnp.int32(0))

        @pl.when(wid == _NW - 1)
        def _tail():
            pltpu.sync_copy(tail_hbm, tailv)
            _scan_tail(sel_idx, chl, tailv, hitrows_v, cnt)

        pltpu.sync_copy(hitrows_v, stage_hbm.at[pl.ds(rowbase, _HITROWS)])


def _k2_body(stage_hbm, out_hbm, selb_v, einfo_v, srclist_v, fetch_v, outT_v,
             gsem, osem):
    wid = lax.axis_index("s") * _NC + lax.axis_index("c")
    iota = _IOTA()
    bb = wid * _BPW

    for t in range(2):
        def tile_step(src_w, carry):
            base = src_w * _WBLK + t * _TBLK
            pltpu.sync_copy(
                stage_hbm.at[pl.ds(base + _HITROWS, _SELROWS)], selb_v
            )
            cnt = plsc.bitcast(selb_v[8, pl.ds(0, 16)], jnp.int32)[0]
            for j in range(_NSELW):
                bv = plsc.bitcast(
                    selb_v[j // 8, pl.ds((j % 8) * 16, 16)], jnp.int32
                )
                pos = j * 16 + iota
                m = (pos < cnt) & (bv >= bb) & (bv < bb + _BPW)
                e = src_w * 1024 + pos
                plsc.store_scatter(
                    einfo_v, [jnp.where(m, bv - bb, 0)], e, mask=m
                )
            return carry

        lax.fori_loop(0, _NW, tile_step, jnp.int32(0))

        def srcfill(j, carry):
            ev = einfo_v[pl.ds(j * 16, 16)]
            srow = (ev >> 10) * _WBLK + t * _TBLK + ((ev & 1023) >> 2)
            srclist_v[pl.ds(j * 16, 16)] = srow
            return carry

        lax.fori_loop(0, _BPW // 16, srcfill, jnp.int32(0))

        hs = [
            pltpu.async_copy(
                stage_hbm.at[srclist_v.at[pl.ds(q * 128, 128)]],
                fetch_v.at[pl.ds(q * 128, 128)],
                gsem,
            )
            for q in range(_BPW // 128)
        ]
        for h in hs:
            h.wait()

        def assemble(j, carry):
            ev = einfo_v[pl.ds(j * 16, 16)]
            sub = ev & 3
            rows = j * 16 + iota
            for c in range(N_EMBED):
                vals = plsc.load_gather(
                    fetch_v, [rows, sub * N_EMBED + c]
                )
                plsc.store_scatter(
                    outT_v, [_splat(c), j * 16 + iota], vals
                )
            return carry

        lax.fori_loop(0, _BPW // 16, assemble, jnp.int32(0))

        pltpu.async_copy(
            outT_v, out_hbm.at[t, :, pl.ds(bb, _BPW)], osem
        ).wait()


_MESH = plsc.VectorSubcoreMesh(core_axis_name="c", subcore_axis_name="s")
_PARAMS = pltpu.CompilerParams(
    use_tc_tiling_on_sc=True, needs_layout_passes=False
)


@functools.partial(
    pl.kernel,
    mesh=_MESH,
    out_type=jax.ShapeDtypeStruct((_STAGE_ROWS, 128), jnp.float32),
    scratch_types=[
        pltpu.VMEM((BATCH,), jnp.int32),
        pltpu.VMEM((_CAP + 16,), jnp.int32),
        pltpu.VMEM((_SELROWS, 128), jnp.float32),
        pltpu.VMEM((80,), jnp.int32),
        pltpu.VMEM((N_EMBED, _CH), jnp.float32),
        pltpu.VMEM((_HITROWS, 128), jnp.float32),
        pltpu.VMEM((16, 128), jnp.float32),
    ],
    compiler_params=_PARAMS,
)
def _skipgram_scan(*refs):
    _k1_body(*refs)


@functools.partial(
    pl.kernel,
    mesh=_MESH,
    out_type=jax.ShapeDtypeStruct((2, N_EMBED, BATCH), jnp.float32),
    scratch_types=[
        pltpu.VMEM((_SELROWS, 128), jnp.float32),
        pltpu.VMEM((_BPW,), jnp.int32),
        pltpu.VMEM((_BPW,), jnp.int32),
        pltpu.VMEM((_BPW, 128), jnp.float32),
        pltpu.VMEM((N_EMBED, _BPW), jnp.float32),
        pltpu.SemaphoreType.DMA,
        pltpu.SemaphoreType.DMA,
    ],
    compiler_params=_PARAMS,
)
def _skipgram_assemble(*refs):
    _k2_body(*refs)


def kernel(input_words, output_words, in_embed_weight, out_embed_weight):
    idx0 = input_words.astype(jnp.int32)
    idx1 = output_words.astype(jnp.int32)
    tail0 = in_embed_weight[_TAIL_LO:].reshape(16, 128)
    tail1 = out_embed_weight[_TAIL_LO:].reshape(16, 128)
    stage = _skipgram_scan(idx0, idx1, in_embed_weight.T, out_embed_weight.T,
                           tail0, tail1)
    outT = _skipgram_assemble(stage)
    return jnp.transpose(outT, (0, 2, 1))


# speedup vs baseline: 1.4134x; 1.4134x over previous
import functools

import jax
import jax.numpy as jnp
from jax import lax
from jax.experimental import pallas as pl
from jax.experimental.pallas import tpu as pltpu
from jax.experimental.pallas import tpu_sc as plsc

N_VOCAB = 1000000
N_EMBED = 32
BATCH = 16384

_INFO = plsc.get_sparse_core_info()
_NC = _INFO.num_cores
_NS = _INFO.num_subcores
_NW = _NC * _NS
_BPW = BATCH // _NW

_NBLK = (N_VOCAB + 127) // 128
_BLK_BASE = _NBLK // _NW
_BLK_EXTRA = _NBLK - _BLK_BASE * _NW
_TAIL_LO = (N_VOCAB // 128) * 128
_CH = 1024
_NCHUNK = 32

_CAP = 1024
_NSELW = _CAP // 16
_HITROWS = _CAP * N_EMBED // 128
_SELROWS = 16
_TBLK = _HITROWS + _SELROWS
_WBLK = 2 * _TBLK
_STAGE_ROWS = _NW * _WBLK

_IOTA = lambda: lax.iota(jnp.int32, 16)


def _splat(x):
    return jnp.full((16,), x, jnp.int32)


def _chunk_select(sel_idx, chl, cnt, start, width):
    iota = _IOTA()

    def win(j, chcnt):
        sv = sel_idx[pl.ds(j * 16, 16)]
        pos = j * 16 + iota
        m = (pos < cnt) & (sv >= start) & (sv < start + width)
        e2 = pos * 1024 + (sv - start)
        lane = plsc.cumsum(m.astype(jnp.int32)) - 1
        plsc.store_scatter(
            chl, [jnp.where(m, chcnt + lane, 0)], e2, mask=m
        )
        return chcnt + plsc.all_reduce_population_count(m)[0]

    return lax.fori_loop(0, _NSELW, win, jnp.int32(0))


def _scan_chunk(sel_idx, chl, block_v, hitrows_v, cnt, start):
    iota = _IOTA()
    chcnt = _chunk_select(sel_idx, chl, cnt, start, _CH)

    def extract(j2, carry):
        ev = chl[pl.ds(j2 * 16, 16)]
        m2 = (j2 * 16 + iota) < chcnt
        iloc = jnp.where(m2, ev & 1023, 0)
        slot = ev >> 10
        for c in range(N_EMBED):
            fp = slot * N_EMBED + c
            vals = plsc.load_gather(block_v, [_splat(c), iloc], mask=m2)
            plsc.store_scatter(
                hitrows_v, [fp >> 7, fp & 127], vals, mask=m2
            )
        return carry

    lax.fori_loop(0, 4, extract, jnp.int32(0))


def _scan_tail(sel_idx, chl, tailv, hitrows_v, cnt):
    iota = _IOTA()
    chcnt = _chunk_select(sel_idx, chl, cnt, jnp.int32(_TAIL_LO),
                          N_VOCAB - _TAIL_LO)

    def extract(j2, carry):
        ev = chl[pl.ds(j2 * 16, 16)]
        m2 = (j2 * 16 + iota) < chcnt
        iloc = jnp.where(m2, ev & 1023, 0)
        slot = ev >> 10
        for c in range(N_EMBED):
            fps = iloc * N_EMBED + c
            fp = slot * N_EMBED + c
            vals = plsc.load_gather(tailv, [fps >> 7, fps & 127], mask=m2)
            plsc.store_scatter(
                hitrows_v, [fp >> 7, fp & 127], vals, mask=m2
            )
        return carry

    lax.fori_loop(0, 4, extract, jnp.int32(0))


def _k1_body(idx0_hbm, idx1_hbm, tab0_hbm, tab1_hbm, tail0_hbm, tail1_hbm,
             stage_hbm,
             idxv, sel_idx, selb2d, chl, block_a, block_b, hitrows_v, tailv,
             gsem):
    wid = lax.axis_index("s") * _NC + lax.axis_index("c")
    iota = _IOTA()

    start_block = wid * _BLK_BASE + lax.min(wid, _BLK_EXTRA)
    nblocks = _BLK_BASE + (wid < _BLK_EXTRA).astype(jnp.int32)
    lo = start_block * 128
    fetch_hi = lax.min(lo + nblocks * 128, jnp.int32(_TAIL_LO))
    hi_sel = lax.min(lo + nblocks * 128, jnp.int32(N_VOCAB))

    for t, (idx_hbm, tab_hbm, tail_hbm) in enumerate(
        ((idx0_hbm, tab0_hbm, tail0_hbm), (idx1_hbm, tab1_hbm, tail1_hbm))
    ):
        pltpu.sync_copy(idx_hbm, idxv)

        def sel_step(k, cnt):
            v = idxv[pl.ds(k * 16, 16)]
            m = (v >= lo) & (v < hi_sel)
            lane = plsc.cumsum(m.astype(jnp.int32)) - 1
            tgt = jnp.where(m, cnt + lane, 0)
            plsc.store_scatter(sel_idx, [tgt], v, mask=m)
            plsc.store_scatter(
                selb2d,
                [tgt >> 7, tgt & 127],
                plsc.bitcast(k * 16 + iota, jnp.float32),
                mask=m,
            )
            return cnt + plsc.all_reduce_population_count(m)[0]

        cnt = lax.fori_loop(0, BATCH // 16, sel_step, jnp.int32(0))

        selb2d[8, pl.ds(0, 16)] = plsc.bitcast(_splat(cnt), jnp.float32)
        rowbase = wid * _WBLK + t * _TBLK
        pltpu.sync_copy(selb2d, stage_hbm.at[pl.ds(rowbase + _HITROWS,
                                                   _SELROWS)])

        def chunk_start(ci):
            s = lax.min(lo + ci * _CH, fetch_hi - _CH)
            return pl.multiple_of(s, 128)

        def fetch(ci, buf):
            return pltpu.async_copy(
                tab_hbm.at[:, pl.ds(chunk_start(ci), _CH)], buf, gsem
            )

        fetch(jnp.int32(0), block_a)

        def chunk_pair(o, carry):
            for par, buf, nbuf in ((0, block_a, block_b),
                                   (1, block_b, block_a)):
                ci = o * 2 + par
                start = chunk_start(ci)
                pltpu.make_async_copy(
                    tab_hbm.at[:, pl.ds(start, _CH)], buf, gsem
                ).wait()

                @pl.when(ci + 1 < _NCHUNK)
                def _prefetch():
                    fetch(ci + 1, nbuf)

                _scan_chunk(sel_idx, chl, buf, hitrows_v, cnt, start)
            return carry

        lax.fori_loop(0, _NCHUNK // 2, chunk_pair, jnp.int32(0))

        @pl.when(wid == _NW - 1)
        def _tail():
            pltpu.sync_copy(tail_hbm, tailv)
            _scan_tail(sel_idx, chl, tailv, hitrows_v, cnt)

        pltpu.sync_copy(hitrows_v, stage_hbm.at[pl.ds(rowbase, _HITROWS)])


def _k2_body(stage_hbm, out_hbm, selb_a, selb_b, einfo_v, srclist_v, fetch_v,
             outT_v, gsem, osem):
    wid = lax.axis_index("s") * _NC + lax.axis_index("c")
    iota = _IOTA()
    bb = wid * _BPW

    for t in range(2):
        def side_ref(src_w):
            base = src_w * _WBLK + t * _TBLK + _HITROWS
            return stage_hbm.at[pl.ds(base, _SELROWS)]

        def scan_side(src_w, selb_v):
            cnt = plsc.bitcast(selb_v[8, pl.ds(0, 16)], jnp.int32)[0]
            for j in range(_NSELW):
                bv = plsc.bitcast(
                    selb_v[j // 8, pl.ds((j % 8) * 16, 16)], jnp.int32
                )
                pos = j * 16 + iota
                m = (pos < cnt) & (bv >= bb) & (bv < bb + _BPW)
                e = src_w * 1024 + pos
                plsc.store_scatter(
                    einfo_v, [jnp.where(m, bv - bb, 0)], e, mask=m
                )

        pltpu.async_copy(side_ref(jnp.int32(0)), selb_a, gsem)

        def tile_pair(o, carry):
            for par, buf, nbuf in ((0, selb_a, selb_b), (1, selb_b, selb_a)):
                src_w = o * 2 + par
                pltpu.make_async_copy(side_ref(src_w), buf, gsem).wait()

                @pl.when(src_w + 1 < _NW)
                def _prefetch():
                    pltpu.async_copy(side_ref(src_w + 1), nbuf, gsem)

                scan_side(src_w, buf)
            return carry

        lax.fori_loop(0, _NW // 2, tile_pair, jnp.int32(0))

        def srcfill(j, carry):
            ev = einfo_v[pl.ds(j * 16, 16)]
            srow = (ev >> 10) * _WBLK + t * _TBLK + ((ev & 1023) >> 2)
            srclist_v[pl.ds(j * 16, 16)] = srow
            return carry

        lax.fori_loop(0, _BPW // 16, srcfill, jnp.int32(0))

        hs = [
            pltpu.async_copy(
                stage_hbm.at[srclist_v.at[pl.ds(q * 128, 128)]],
                fetch_v.at[pl.ds(q * 128, 128)],
                gsem,
            )
            for q in range(_BPW // 128)
        ]
        for h in hs:
            h.wait()

        def assemble(j, carry):
            ev = einfo_v[pl.ds(j * 16, 16)]
            sub = ev & 3
            rows = j * 16 + iota
            for c in range(N_EMBED):
                vals = plsc.load_gather(
                    fetch_v, [rows, sub * N_EMBED + c]
                )
                plsc.store_scatter(
                    outT_v, [_splat(c), j * 16 + iota], vals
                )
            return carry

        lax.fori_loop(0, _BPW // 16, assemble, jnp.int32(0))

        pltpu.async_copy(
            outT_v, out_hbm.at[t, :, pl.ds(bb, _BPW)], osem
        ).wait()


_MESH = plsc.VectorSubcoreMesh(core_axis_name="c", subcore_axis_name="s")
_PARAMS = pltpu.CompilerParams(
    use_tc_tiling_on_sc=True, needs_layout_passes=False
)


@functools.partial(
    pl.kernel,
    mesh=_MESH,
    out_type=jax.ShapeDtypeStruct((_STAGE_ROWS, 128), jnp.float32),
    scratch_types=[
        pltpu.VMEM((BATCH,), jnp.int32),
        pltpu.VMEM((_CAP + 16,), jnp.int32),
        pltpu.VMEM((_SELROWS, 128), jnp.float32),
        pltpu.VMEM((80,), jnp.int32),
        pltpu.VMEM((N_EMBED, _CH), jnp.float32),
        pltpu.VMEM((N_EMBED, _CH), jnp.float32),
        pltpu.VMEM((_HITROWS, 128), jnp.float32),
        pltpu.VMEM((16, 128), jnp.float32),
        pltpu.SemaphoreType.DMA,
    ],
    compiler_params=_PARAMS,
)
def _skipgram_scan(*refs):
    _k1_body(*refs)


@functools.partial(
    pl.kernel,
    mesh=_MESH,
    out_type=jax.ShapeDtypeStruct((2, N_EMBED, BATCH), jnp.float32),
    scratch_types=[
        pltpu.VMEM((_SELROWS, 128), jnp.float32),
        pltpu.VMEM((_SELROWS, 128), jnp.float32),
        pltpu.VMEM((_BPW,), jnp.int32),
        pltpu.VMEM((_BPW,), jnp.int32),
        pltpu.VMEM((_BPW, 128), jnp.float32),
        pltpu.VMEM((N_EMBED, _BPW), jnp.float32),
        pltpu.SemaphoreType.DMA,
        pltpu.SemaphoreType.DMA,
    ],
    compiler_params=_PARAMS,
)
def _skipgram_assemble(*refs):
    _k2_body(*refs)


def kernel(input_words, output_words, in_embed_weight, out_embed_weight):
    idx0 = input_words.astype(jnp.int32)
    idx1 = output_words.astype(jnp.int32)
    tail0 = in_embed_weight[_TAIL_LO:].reshape(16, 128)
    tail1 = out_embed_weight[_TAIL_LO:].reshape(16, 128)
    stage = _skipgram_scan(idx0, idx1, in_embed_weight.T, out_embed_weight.T,
                           tail0, tail1)
    outT = _skipgram_assemble(stage)
    return jnp.transpose(outT, (0, 2, 1))


# speedup vs baseline: 1.4272x; 1.0098x over previous
import functools

import jax
import jax.numpy as jnp
from jax import lax
from jax.experimental import pallas as pl
from jax.experimental.pallas import tpu as pltpu
from jax.experimental.pallas import tpu_sc as plsc

N_VOCAB = 1000000
N_EMBED = 32
BATCH = 16384

_INFO = plsc.get_sparse_core_info()
_NC = _INFO.num_cores
_NS = _INFO.num_subcores
_NW = _NC * _NS
_BPW = BATCH // _NW

_NBLK = (N_VOCAB + 127) // 128
_BLK_BASE = _NBLK // _NW
_BLK_EXTRA = _NBLK - _BLK_BASE * _NW
_TAIL_LO = (N_VOCAB // 128) * 128
_CH = 1024
_NCHUNK = 32

_CAP = 1024
_NSELW = _CAP // 16
_HITROWS = _CAP * N_EMBED // 128
_SELROWS = 16
_TBLK = _HITROWS + _SELROWS
_WBLK = 2 * _TBLK
_STAGE_ROWS = _NW * _WBLK

_IOTA = lambda: lax.iota(jnp.int32, 16)


def _splat(x):
    return jnp.full((16,), x, jnp.int32)


def _chunk_select(sel_idx, chl, cnt, start, width):
    iota = _IOTA()

    def win(j, chcnt):
        sv = sel_idx[pl.ds(j * 16, 16)]
        pos = j * 16 + iota
        m = (pos < cnt) & (sv >= start) & (sv < start + width)
        e2 = pos * 1024 + (sv - start)
        lane = plsc.cumsum(m.astype(jnp.int32)) - 1
        plsc.store_scatter(
            chl, [jnp.where(m, chcnt + lane, 0)], e2, mask=m
        )
        return chcnt + plsc.all_reduce_population_count(m)[0]

    return lax.fori_loop(0, (cnt + 15) >> 4, win, jnp.int32(0))


def _scan_chunk(sel_idx, chl, block_v, hitrows_v, cnt, start):
    iota = _IOTA()
    chcnt = _chunk_select(sel_idx, chl, cnt, start, _CH)

    def extract(j2, carry):
        ev = chl[pl.ds(j2 * 16, 16)]
        m2 = (j2 * 16 + iota) < chcnt
        iloc = jnp.where(m2, ev & 1023, 0)
        slot = ev >> 10
        for c in range(N_EMBED):
            fp = slot * N_EMBED + c
            vals = plsc.load_gather(block_v, [_splat(c), iloc], mask=m2)
            plsc.store_scatter(
                hitrows_v, [fp >> 7, fp & 127], vals, mask=m2
            )
        return carry

    lax.fori_loop(0, (chcnt + 15) >> 4, extract, jnp.int32(0))


def _scan_tail(sel_idx, chl, tailv, hitrows_v, cnt):
    iota = _IOTA()
    chcnt = _chunk_select(sel_idx, chl, cnt, jnp.int32(_TAIL_LO),
                          N_VOCAB - _TAIL_LO)

    def extract(j2, carry):
        ev = chl[pl.ds(j2 * 16, 16)]
        m2 = (j2 * 16 + iota) < chcnt
        iloc = jnp.where(m2, ev & 1023, 0)
        slot = ev >> 10
        for c in range(N_EMBED):
            fps = iloc * N_EMBED + c
            fp = slot * N_EMBED + c
            vals = plsc.load_gather(tailv, [fps >> 7, fps & 127], mask=m2)
            plsc.store_scatter(
                hitrows_v, [fp >> 7, fp & 127], vals, mask=m2
            )
        return carry

    lax.fori_loop(0, 4, extract, jnp.int32(0))


def _k1_body(idx0_hbm, idx1_hbm, tab0_hbm, tab1_hbm, tail0_hbm, tail1_hbm,
             stage_hbm,
             idxv, sel_idx, selb2d, chl, block_a, block_b, hitrows_v, tailv,
             gsem):
    wid = lax.axis_index("s") * _NC + lax.axis_index("c")
    iota = _IOTA()

    start_block = wid * _BLK_BASE + lax.min(wid, _BLK_EXTRA)
    nblocks = _BLK_BASE + (wid < _BLK_EXTRA).astype(jnp.int32)
    lo = start_block * 128
    fetch_hi = lax.min(lo + nblocks * 128, jnp.int32(_TAIL_LO))
    hi_sel = lax.min(lo + nblocks * 128, jnp.int32(N_VOCAB))

    for t, (idx_hbm, tab_hbm, tail_hbm) in enumerate(
        ((idx0_hbm, tab0_hbm, tail0_hbm), (idx1_hbm, tab1_hbm, tail1_hbm))
    ):
        pltpu.sync_copy(idx_hbm, idxv)

        def sel_step(k, cnt):
            v = idxv[pl.ds(k * 16, 16)]
            m = (v >= lo) & (v < hi_sel)
            lane = plsc.cumsum(m.astype(jnp.int32)) - 1
            tgt = jnp.where(m, cnt + lane, 0)
            plsc.store_scatter(sel_idx, [tgt], v, mask=m)
            plsc.store_scatter(
                selb2d,
                [tgt >> 7, tgt & 127],
                plsc.bitcast(k * 16 + iota, jnp.float32),
                mask=m,
            )
            return cnt + plsc.all_reduce_population_count(m)[0]

        cnt = lax.fori_loop(0, BATCH // 16, sel_step, jnp.int32(0))

        selb2d[8, pl.ds(0, 16)] = plsc.bitcast(_splat(cnt), jnp.float32)
        rowbase = wid * _WBLK + t * _TBLK
        pltpu.sync_copy(selb2d, stage_hbm.at[pl.ds(rowbase + _HITROWS,
                                                   _SELROWS)])

        def chunk_start(ci):
            s = lax.min(lo + ci * _CH, fetch_hi - _CH)
            return pl.multiple_of(s, 128)

        def fetch(ci, buf):
            return pltpu.async_copy(
                tab_hbm.at[:, pl.ds(chunk_start(ci), _CH)], buf, gsem
            )

        fetch(jnp.int32(0), block_a)

        def chunk_pair(o, carry):
            for par, buf, nbuf in ((0, block_a, block_b),
                                   (1, block_b, block_a)):
                ci = o * 2 + par
                start = chunk_start(ci)
                pltpu.make_async_copy(
                    tab_hbm.at[:, pl.ds(start, _CH)], buf, gsem
                ).wait()

                @pl.when(ci + 1 < _NCHUNK)
                def _prefetch():
                    fetch(ci + 1, nbuf)

                _scan_chunk(sel_idx, chl, buf, hitrows_v, cnt, start)
            return carry

        lax.fori_loop(0, _NCHUNK // 2, chunk_pair, jnp.int32(0))

        @pl.when(wid == _NW - 1)
        def _tail():
            pltpu.sync_copy(tail_hbm, tailv)
            _scan_tail(sel_idx, chl, tailv, hitrows_v, cnt)

        pltpu.sync_copy(hitrows_v, stage_hbm.at[pl.ds(rowbase, _HITROWS)])


def _k2_body(stage_hbm, out_hbm, selb_a, selb_b, einfo_v, srclist_v, fetch_v,
             outT_v, gsem, osem):
    wid = lax.axis_index("s") * _NC + lax.axis_index("c")
    iota = _IOTA()
    bb = wid * _BPW

    for t in range(2):
        def side_ref(src_w):
            base = src_w * _WBLK + t * _TBLK + _HITROWS
            return stage_hbm.at[pl.ds(base, _SELROWS)]

        def scan_side(src_w, selb_v):
            cnt = plsc.bitcast(selb_v[8, pl.ds(0, 16)], jnp.int32)[0]
            for j in range(_NSELW):
                bv = plsc.bitcast(
                    selb_v[j // 8, pl.ds((j % 8) * 16, 16)], jnp.int32
                )
                pos = j * 16 + iota
                m = (pos < cnt) & (bv >= bb) & (bv < bb + _BPW)
                e = src_w * 1024 + pos
                plsc.store_scatter(
                    einfo_v, [jnp.where(m, bv - bb, 0)], e, mask=m
                )

        pltpu.async_copy(side_ref(jnp.int32(0)), selb_a, gsem)

        def tile_pair(o, carry):
            for par, buf, nbuf in ((0, selb_a, selb_b), (1, selb_b, selb_a)):
                src_w = o * 2 + par
                pltpu.make_async_copy(side_ref(src_w), buf, gsem).wait()

                @pl.when(src_w + 1 < _NW)
                def _prefetch():
                    pltpu.async_copy(side_ref(src_w + 1), nbuf, gsem)

                scan_side(src_w, buf)
            return carry

        lax.fori_loop(0, _NW // 2, tile_pair, jnp.int32(0))

        def srcfill(j, carry):
            ev = einfo_v[pl.ds(j * 16, 16)]
            srow = (ev >> 10) * _WBLK + t * _TBLK + ((ev & 1023) >> 2)
            srclist_v[pl.ds(j * 16, 16)] = srow
            return carry

        lax.fori_loop(0, _BPW // 16, srcfill, jnp.int32(0))

        hs = [
            pltpu.async_copy(
                stage_hbm.at[srclist_v.at[pl.ds(q * 128, 128)]],
                fetch_v.at[pl.ds(q * 128, 128)],
                gsem,
            )
            for q in range(_BPW // 128)
        ]
        for h in hs:
            h.wait()

        def assemble(j, carry):
            ev = einfo_v[pl.ds(j * 16, 16)]
            sub = ev & 3
            rows = j * 16 + iota
            for c in range(N_EMBED):
                vals = plsc.load_gather(
                    fetch_v, [rows, sub * N_EMBED + c]
                )
                plsc.store_scatter(
                    outT_v, [_splat(c), j * 16 + iota], vals
                )
            return carry

        lax.fori_loop(0, _BPW // 16, assemble, jnp.int32(0))

        pltpu.async_copy(
            outT_v, out_hbm.at[t, :, pl.ds(bb, _BPW)], osem
        ).wait()


_MESH = plsc.VectorSubcoreMesh(core_axis_name="c", subcore_axis_name="s")
_PARAMS = pltpu.CompilerParams(
    use_tc_tiling_on_sc=True, needs_layout_passes=False
)


@functools.partial(
    pl.kernel,
    mesh=_MESH,
    out_type=jax.ShapeDtypeStruct((_STAGE_ROWS, 128), jnp.float32),
    scratch_types=[
        pltpu.VMEM((BATCH,), jnp.int32),
        pltpu.VMEM((_CAP + 16,), jnp.int32),
        pltpu.VMEM((_SELROWS, 128), jnp.float32),
        pltpu.VMEM((80,), jnp.int32),
        pltpu.VMEM((N_EMBED, _CH), jnp.float32),
        pltpu.VMEM((N_EMBED, _CH), jnp.float32),
        pltpu.VMEM((_HITROWS, 128), jnp.float32),
        pltpu.VMEM((16, 128), jnp.float32),
        pltpu.SemaphoreType.DMA,
    ],
    compiler_params=_PARAMS,
)
def _skipgram_scan(*refs):
    _k1_body(*refs)


@functools.partial(
    pl.kernel,
    mesh=_MESH,
    out_type=jax.ShapeDtypeStruct((2, N_EMBED, BATCH), jnp.float32),
    scratch_types=[
        pltpu.VMEM((_SELROWS, 128), jnp.float32),
        pltpu.VMEM((_SELROWS, 128), jnp.float32),
        pltpu.VMEM((_BPW,), jnp.int32),
        pltpu.VMEM((_BPW,), jnp.int32),
        pltpu.VMEM((_BPW, 128), jnp.float32),
        pltpu.VMEM((N_EMBED, _BPW), jnp.float32),
        pltpu.SemaphoreType.DMA,
        pltpu.SemaphoreType.DMA,
    ],
    compiler_params=_PARAMS,
)
def _skipgram_assemble(*refs):
    _k2_body(*refs)


def kernel(input_words, output_words, in_embed_weight, out_embed_weight):
    idx0 = input_words.astype(jnp.int32)
    idx1 = output_words.astype(jnp.int32)
    tail0 = in_embed_weight[_TAIL_LO:].reshape(16, 128)
    tail1 = out_embed_weight[_TAIL_LO:].reshape(16, 128)
    stage = _skipgram_scan(idx0, idx1, in_embed_weight.T, out_embed_weight.T,
                           tail0, tail1)
    outT = _skipgram_assemble(stage)
    return jnp.transpose(outT, (0, 2, 1))


# speedup vs baseline: 1.5406x; 1.0794x over previous
import functools

import jax
import jax.numpy as jnp
from jax import lax
from jax.experimental import pallas as pl
from jax.experimental.pallas import tpu as pltpu
from jax.experimental.pallas import tpu_sc as plsc

N_VOCAB = 1000000
N_EMBED = 32
BATCH = 16384

_INFO = plsc.get_sparse_core_info()
_NC = _INFO.num_cores
_NS = _INFO.num_subcores
_NW = _NC * _NS
_BPW = BATCH // _NW

_NBLK = (N_VOCAB + 127) // 128
_BLK_BASE = _NBLK // _NW
_BLK_EXTRA = _NBLK - _BLK_BASE * _NW
_TAIL_LO = (N_VOCAB // 128) * 128
_CH = 1024
_NCHUNK = 32

_CAP = 1024
_NSELW = _CAP // 16
_HITROWS = _CAP * N_EMBED // 128
_SELROWS = 16
_SIDE0 = _NW * 2 * _HITROWS
_STAGE_ROWS = _SIDE0 + _NW * 2 * _SELROWS

_IOTA = lambda: lax.iota(jnp.int32, 16)


def _splat(x):
    return jnp.full((16,), x, jnp.int32)


def _chunk_select(sel_idx, chl, cnt, start, width):
    iota = _IOTA()

    def win(j, chcnt):
        sv = sel_idx[pl.ds(j * 16, 16)]
        pos = j * 16 + iota
        m = (pos < cnt) & (sv >= start) & (sv < start + width)
        e2 = pos * 1024 + (sv - start)
        lane = plsc.cumsum(m.astype(jnp.int32)) - 1
        plsc.store_scatter(
            chl, [jnp.where(m, chcnt + lane, 0)], e2, mask=m
        )
        return chcnt + plsc.all_reduce_population_count(m)[0]

    return lax.fori_loop(0, (cnt + 15) >> 4, win, jnp.int32(0))


def _scan_chunk(sel_idx, chl, block_v, hitrows_v, cnt, start):
    iota = _IOTA()
    chcnt = _chunk_select(sel_idx, chl, cnt, start, _CH)

    def extract(j2, carry):
        ev = chl[pl.ds(j2 * 16, 16)]
        m2 = (j2 * 16 + iota) < chcnt
        iloc = jnp.where(m2, ev & 1023, 0)
        slot = ev >> 10
        for c in range(N_EMBED):
            fp = slot * N_EMBED + c
            vals = plsc.load_gather(block_v, [_splat(c), iloc], mask=m2)
            plsc.store_scatter(
                hitrows_v, [fp >> 7, fp & 127], vals, mask=m2
            )
        return carry

    lax.fori_loop(0, (chcnt + 15) >> 4, extract, jnp.int32(0))


def _scan_tail(sel_idx, chl, tailv, hitrows_v, cnt):
    iota = _IOTA()
    chcnt = _chunk_select(sel_idx, chl, cnt, jnp.int32(_TAIL_LO),
                          N_VOCAB - _TAIL_LO)

    def extract(j2, carry):
        ev = chl[pl.ds(j2 * 16, 16)]
        m2 = (j2 * 16 + iota) < chcnt
        iloc = jnp.where(m2, ev & 1023, 0)
        slot = ev >> 10
        for c in range(N_EMBED):
            fps = iloc * N_EMBED + c
            fp = slot * N_EMBED + c
            vals = plsc.load_gather(tailv, [fps >> 7, fps & 127], mask=m2)
            plsc.store_scatter(
                hitrows_v, [fp >> 7, fp & 127], vals, mask=m2
            )
        return carry

    lax.fori_loop(0, 4, extract, jnp.int32(0))


def _k1_body(idx0_hbm, idx1_hbm, tab0_hbm, tab1_hbm, tail0_hbm, tail1_hbm,
             stage_hbm,
             idxv, sel_idx, selb2d, chl, block_a, block_b, hitrows_v, tailv,
             gsem):
    wid = lax.axis_index("s") * _NC + lax.axis_index("c")
    iota = _IOTA()

    start_block = wid * _BLK_BASE + lax.min(wid, _BLK_EXTRA)
    nblocks = _BLK_BASE + (wid < _BLK_EXTRA).astype(jnp.int32)
    lo = start_block * 128
    fetch_hi = lax.min(lo + nblocks * 128, jnp.int32(_TAIL_LO))
    hi_sel = lax.min(lo + nblocks * 128, jnp.int32(N_VOCAB))

    for t, (idx_hbm, tab_hbm, tail_hbm) in enumerate(
        ((idx0_hbm, tab0_hbm, tail0_hbm), (idx1_hbm, tab1_hbm, tail1_hbm))
    ):
        pltpu.sync_copy(idx_hbm, idxv)

        def sel_step(k, cnt):
            v = idxv[pl.ds(k * 16, 16)]
            m = (v >= lo) & (v < hi_sel)
            lane = plsc.cumsum(m.astype(jnp.int32)) - 1
            tgt = jnp.where(m, cnt + lane, 0)
            plsc.store_scatter(sel_idx, [tgt], v, mask=m)
            plsc.store_scatter(
                selb2d,
                [tgt >> 7, tgt & 127],
                plsc.bitcast(k * 16 + iota, jnp.float32),
                mask=m,
            )
            return cnt + plsc.all_reduce_population_count(m)[0]

        cnt = lax.fori_loop(0, BATCH // 16, sel_step, jnp.int32(0))

        selb2d[8, pl.ds(0, 16)] = plsc.bitcast(_splat(cnt), jnp.float32)
        rowbase = (wid * 2 + t) * _HITROWS
        siderow = _SIDE0 + (t * _NW + wid) * _SELROWS
        pltpu.sync_copy(selb2d, stage_hbm.at[pl.ds(siderow, _SELROWS)])

        def chunk_start(ci):
            s = lax.min(lo + ci * _CH, fetch_hi - _CH)
            return pl.multiple_of(s, 128)

        def fetch(ci, buf):
            return pltpu.async_copy(
                tab_hbm.at[:, pl.ds(chunk_start(ci), _CH)], buf, gsem
            )

        fetch(jnp.int32(0), block_a)

        def chunk_pair(o, carry):
            for par, buf, nbuf in ((0, block_a, block_b),
                                   (1, block_b, block_a)):
                ci = o * 2 + par
                start = chunk_start(ci)
                pltpu.make_async_copy(
                    tab_hbm.at[:, pl.ds(start, _CH)], buf, gsem
                ).wait()

                @pl.when(ci + 1 < _NCHUNK)
                def _prefetch():
                    fetch(ci + 1, nbuf)

                _scan_chunk(sel_idx, chl, buf, hitrows_v, cnt, start)
            return carry

        lax.fori_loop(0, _NCHUNK // 2, chunk_pair, jnp.int32(0))

        @pl.when(wid == _NW - 1)
        def _tail():
            pltpu.sync_copy(tail_hbm, tailv)
            _scan_tail(sel_idx, chl, tailv, hitrows_v, cnt)

        pltpu.sync_copy(hitrows_v, stage_hbm.at[pl.ds(rowbase, _HITROWS)])


def _k2_body(stage_hbm, out_hbm, buf_a, buf_b, einfo_v, srclist_v, outT_v,
             gsem, osem):
    wid = lax.axis_index("s") * _NC + lax.axis_index("c")
    iota = _IOTA()
    bb = wid * _BPW
    half = _NW // 2 * _SELROWS

    for t in range(2):
        side0 = _SIDE0 + t * _NW * _SELROWS
        ha = pltpu.async_copy(
            stage_hbm.at[pl.ds(side0, half)], buf_a, gsem
        )
        hb = pltpu.async_copy(
            stage_hbm.at[pl.ds(side0 + half, half)], buf_b, gsem
        )

        def scan_half(buf, w0):
            def scan_tile(wloc, carry):
                src_w = w0 + wloc
                base = wloc * _SELROWS
                cnt = plsc.bitcast(
                    plsc.load_gather(buf, [_splat(base + 8), iota]),
                    jnp.int32,
                )[0]

                def win(j, carry2):
                    bv = plsc.bitcast(
                        plsc.load_gather(
                            buf, [_splat(base + (j >> 3)),
                                  (j & 7) * 16 + iota]
                        ),
                        jnp.int32,
                    )
                    pos = j * 16 + iota
                    m = (pos < cnt) & (bv >= bb) & (bv < bb + _BPW)
                    e = src_w * 1024 + pos
                    plsc.store_scatter(
                        einfo_v, [jnp.where(m, bv - bb, 0)], e, mask=m
                    )
                    return carry2

                lax.fori_loop(0, (cnt + 15) >> 4, win, jnp.int32(0))
                return carry

            lax.fori_loop(0, _NW // 2, scan_tile, jnp.int32(0))

        ha.wait()
        scan_half(buf_a, 0)
        hb.wait()
        scan_half(buf_b, _NW // 2)

        def srcfill(j, carry):
            ev = einfo_v[pl.ds(j * 16, 16)]
            srow = ((ev >> 10) * 2 + t) * _HITROWS + ((ev & 1023) >> 2)
            srclist_v[pl.ds(j * 16, 16)] = srow
            return carry

        lax.fori_loop(0, _BPW // 16, srcfill, jnp.int32(0))

        hs = [
            pltpu.async_copy(
                stage_hbm.at[srclist_v.at[pl.ds(q * 128, 128)]],
                (buf_a if q < 2 else buf_b).at[pl.ds((q % 2) * 128, 128)],
                gsem,
            )
            for q in range(_BPW // 128)
        ]
        for h in hs:
            h.wait()

        def assemble(j, carry):
            ev = einfo_v[pl.ds(j * 16, 16)]
            sub = ev & 3
            rows = (j * 16 + iota) & 255
            buf = buf_a
            for c in range(N_EMBED):
                vals = plsc.load_gather(buf, [rows, sub * N_EMBED + c])
                plsc.store_scatter(
                    outT_v, [_splat(c), j * 16 + iota], vals
                )
            return carry

        def assemble_b(j, carry):
            ev = einfo_v[pl.ds(j * 16, 16)]
            sub = ev & 3
            rows = (j * 16 + iota) & 255
            for c in range(N_EMBED):
                vals = plsc.load_gather(buf_b, [rows, sub * N_EMBED + c])
                plsc.store_scatter(
                    outT_v, [_splat(c), j * 16 + iota], vals
                )
            return carry

        lax.fori_loop(0, _BPW // 32, assemble, jnp.int32(0))
        lax.fori_loop(_BPW // 32, _BPW // 16, assemble_b, jnp.int32(0))

        pltpu.async_copy(
            outT_v, out_hbm.at[t, :, pl.ds(bb, _BPW)], osem
        ).wait()


_MESH = plsc.VectorSubcoreMesh(core_axis_name="c", subcore_axis_name="s")
_PARAMS = pltpu.CompilerParams(
    use_tc_tiling_on_sc=True, needs_layout_passes=False
)


@functools.partial(
    pl.kernel,
    mesh=_MESH,
    out_type=jax.ShapeDtypeStruct((_STAGE_ROWS, 128), jnp.float32),
    scratch_types=[
        pltpu.VMEM((BATCH,), jnp.int32),
        pltpu.VMEM((_CAP + 16,), jnp.int32),
        pltpu.VMEM((_SELROWS, 128), jnp.float32),
        pltpu.VMEM((80,), jnp.int32),
        pltpu.VMEM((N_EMBED, _CH), jnp.float32),
        pltpu.VMEM((N_EMBED, _CH), jnp.float32),
        pltpu.VMEM((_HITROWS, 128), jnp.float32),
        pltpu.VMEM((16, 128), jnp.float32),
        pltpu.SemaphoreType.DMA,
    ],
    compiler_params=_PARAMS,
)
def _skipgram_scan(*refs):
    _k1_body(*refs)


@functools.partial(
    pl.kernel,
    mesh=_MESH,
    out_type=jax.ShapeDtypeStruct((2, N_EMBED, BATCH), jnp.float32),
    scratch_types=[
        pltpu.VMEM((_NW // 2 * _SELROWS, 128), jnp.float32),
        pltpu.VMEM((_NW // 2 * _SELROWS, 128), jnp.float32),
        pltpu.VMEM((_BPW,), jnp.int32),
        pltpu.VMEM((_BPW,), jnp.int32),
        pltpu.VMEM((N_EMBED, _BPW), jnp.float32),
        pltpu.SemaphoreType.DMA,
        pltpu.SemaphoreType.DMA,
    ],
    compiler_params=_PARAMS,
)
def _skipgram_assemble(*refs):
    _k2_body(*refs)


def kernel(input_words, output_words, in_embed_weight, out_embed_weight):
    idx0 = input_words.astype(jnp.int32)
    idx1 = output_words.astype(jnp.int32)
    tail0 = in_embed_weight[_TAIL_LO:].reshape(16, 128)
    tail1 = out_embed_weight[_TAIL_LO:].reshape(16, 128)
    stage = _skipgram_scan(idx0, idx1, in_embed_weight.T, out_embed_weight.T,
                           tail0, tail1)
    outT = _skipgram_assemble(stage)
    return jnp.transpose(outT, (0, 2, 1))


# speedup vs baseline: 1.5613x; 1.0134x over previous
import functools

import jax
import jax.numpy as jnp
from jax import lax
from jax.experimental import pallas as pl
from jax.experimental.pallas import tpu as pltpu
from jax.experimental.pallas import tpu_sc as plsc

N_VOCAB = 1000000
N_EMBED = 32
BATCH = 16384

_INFO = plsc.get_sparse_core_info()
_NC = _INFO.num_cores
_NS = _INFO.num_subcores
_NW = _NC * _NS
_BPW = BATCH // _NW

_NBLK = (N_VOCAB + 127) // 128
_BLK_BASE = _NBLK // _NW
_BLK_EXTRA = _NBLK - _BLK_BASE * _NW
_TAIL_LO = (N_VOCAB // 128) * 128
_CH = 1024
_NCHUNK = 32

_CAP = 1024
_NSELW = _CAP // 16
_HITROWS = _CAP * N_EMBED // 128
_SELROWS = 16
_SIDE0 = _NW * 2 * _HITROWS
_STAGE_ROWS = _SIDE0 + _NW * 2 * _SELROWS

_IOTA = lambda: lax.iota(jnp.int32, 16)


def _splat(x):
    return jnp.full((16,), x, jnp.int32)


def _chunk_select(sel_idx, chl, cnt, start, width):
    iota = _IOTA()

    def win(j, chcnt):
        sv = sel_idx[pl.ds(j * 16, 16)]
        pos = j * 16 + iota
        m = (pos < cnt) & (sv >= start) & (sv < start + width)
        e2 = pos * 1024 + (sv - start)
        lane = plsc.cumsum(m.astype(jnp.int32)) - 1
        plsc.store_scatter(
            chl, [jnp.where(m, chcnt + lane, 0)], e2, mask=m
        )
        return chcnt + plsc.all_reduce_population_count(m)[0]

    return lax.fori_loop(0, (cnt + 15) >> 4, win, jnp.int32(0))


def _scan_chunk(sel_idx, chl, block_v, hitrows_v, cnt, start):
    iota = _IOTA()
    chcnt = _chunk_select(sel_idx, chl, cnt, start, _CH)

    def extract(j2, carry):
        ev = chl[pl.ds(j2 * 16, 16)]
        m2 = (j2 * 16 + iota) < chcnt
        iloc = jnp.where(m2, ev & 1023, 0)
        slot = ev >> 10
        for c in range(N_EMBED):
            fp = slot * N_EMBED + c
            vals = plsc.load_gather(block_v, [_splat(c), iloc], mask=m2)
            plsc.store_scatter(
                hitrows_v, [fp >> 7, fp & 127], vals, mask=m2
            )
        return carry

    lax.fori_loop(0, (chcnt + 15) >> 4, extract, jnp.int32(0))


def _scan_tail(sel_idx, chl, tailv, hitrows_v, cnt):
    iota = _IOTA()
    chcnt = _chunk_select(sel_idx, chl, cnt, jnp.int32(_TAIL_LO),
                          N_VOCAB - _TAIL_LO)

    def extract(j2, carry):
        ev = chl[pl.ds(j2 * 16, 16)]
        m2 = (j2 * 16 + iota) < chcnt
        iloc = jnp.where(m2, ev & 1023, 0)
        slot = ev >> 10
        for c in range(N_EMBED):
            fps = iloc * N_EMBED + c
            fp = slot * N_EMBED + c
            vals = plsc.load_gather(tailv, [fps >> 7, fps & 127], mask=m2)
            plsc.store_scatter(
                hitrows_v, [fp >> 7, fp & 127], vals, mask=m2
            )
        return carry

    lax.fori_loop(0, 4, extract, jnp.int32(0))


def _k1_body(idx0_hbm, idx1_hbm, tab0_hbm, tab1_hbm, tail0_hbm, tail1_hbm,
             stage_hbm,
             idxv, sel_idx, selb2d, chl, block_a, block_b, hitrows_v, tailv,
             gsem):
    wid = lax.axis_index("s") * _NC + lax.axis_index("c")
    iota = _IOTA()

    start_block = wid * _BLK_BASE + lax.min(wid, _BLK_EXTRA)
    nblocks = _BLK_BASE + (wid < _BLK_EXTRA).astype(jnp.int32)
    lo = start_block * 128
    fetch_hi = lax.min(lo + nblocks * 128, jnp.int32(_TAIL_LO))
    hi_sel = lax.min(lo + nblocks * 128, jnp.int32(N_VOCAB))

    for t, (idx_hbm, tab_hbm, tail_hbm) in enumerate(
        ((idx0_hbm, tab0_hbm, tail0_hbm), (idx1_hbm, tab1_hbm, tail1_hbm))
    ):
        pltpu.sync_copy(idx_hbm, idxv)

        def chunk_start(ci):
            s = lax.min(lo + ci * _CH, fetch_hi - _CH)
            return pl.multiple_of(s, 128)

        def fetch(ci, buf):
            return pltpu.async_copy(
                tab_hbm.at[:, pl.ds(chunk_start(ci), _CH)], buf, gsem
            )

        fetch(jnp.int32(0), block_a)

        def sel_step(k, cnt):
            v = idxv[pl.ds(k * 16, 16)]
            m = (v >= lo) & (v < hi_sel)
            lane = plsc.cumsum(m.astype(jnp.int32)) - 1
            tgt = jnp.where(m, cnt + lane, 0)
            plsc.store_scatter(sel_idx, [tgt], v, mask=m)
            plsc.store_scatter(
                selb2d,
                [tgt >> 7, tgt & 127],
                plsc.bitcast(k * 16 + iota, jnp.float32),
                mask=m,
            )
            return cnt + plsc.all_reduce_population_count(m)[0]

        cnt = lax.fori_loop(0, BATCH // 16, sel_step, jnp.int32(0))

        selb2d[8, pl.ds(0, 16)] = plsc.bitcast(_splat(cnt), jnp.float32)
        rowbase = (wid * 2 + t) * _HITROWS
        siderow = _SIDE0 + (t * _NW + wid) * _SELROWS
        pltpu.sync_copy(selb2d, stage_hbm.at[pl.ds(siderow, _SELROWS)])

        def chunk_pair(o, carry):
            for par, buf, nbuf in ((0, block_a, block_b),
                                   (1, block_b, block_a)):
                ci = o * 2 + par
                start = chunk_start(ci)
                pltpu.make_async_copy(
                    tab_hbm.at[:, pl.ds(start, _CH)], buf, gsem
                ).wait()

                @pl.when(ci + 1 < _NCHUNK)
                def _prefetch():
                    fetch(ci + 1, nbuf)

                _scan_chunk(sel_idx, chl, buf, hitrows_v, cnt, start)
            return carry

        lax.fori_loop(0, _NCHUNK // 2, chunk_pair, jnp.int32(0))

        @pl.when(wid == _NW - 1)
        def _tail():
            pltpu.sync_copy(tail_hbm, tailv)
            _scan_tail(sel_idx, chl, tailv, hitrows_v, cnt)

        pltpu.sync_copy(hitrows_v, stage_hbm.at[pl.ds(rowbase, _HITROWS)])


def _k2_body(stage_hbm, out_hbm, buf_a, buf_b, einfo_v, srclist_v, outT_v,
             gsem, osem):
    wid = lax.axis_index("s") * _NC + lax.axis_index("c")
    iota = _IOTA()
    bb = wid * _BPW
    half = _NW // 2 * _SELROWS

    for t in range(2):
        side0 = _SIDE0 + t * _NW * _SELROWS
        ha = pltpu.async_copy(
            stage_hbm.at[pl.ds(side0, half)], buf_a, gsem
        )
        hb = pltpu.async_copy(
            stage_hbm.at[pl.ds(side0 + half, half)], buf_b, gsem
        )

        def scan_half(buf, w0):
            def scan_tile(wloc, carry):
                src_w = w0 + wloc
                base = wloc * _SELROWS
                cnt = plsc.bitcast(
                    plsc.load_gather(buf, [_splat(base + 8), iota]),
                    jnp.int32,
                )[0]

                def win(j, carry2):
                    bv = plsc.bitcast(
                        plsc.load_gather(
                            buf, [_splat(base + (j >> 3)),
                                  (j & 7) * 16 + iota]
                        ),
                        jnp.int32,
                    )
                    pos = j * 16 + iota
                    m = (pos < cnt) & (bv >= bb) & (bv < bb + _BPW)
                    e = src_w * 1024 + pos
                    plsc.store_scatter(
                        einfo_v, [jnp.where(m, bv - bb, 0)], e, mask=m
                    )
                    return carry2

                lax.fori_loop(0, (cnt + 15) >> 4, win, jnp.int32(0))
                return carry

            lax.fori_loop(0, _NW // 2, scan_tile, jnp.int32(0))

        ha.wait()
        scan_half(buf_a, 0)
        hb.wait()
        scan_half(buf_b, _NW // 2)

        def srcfill(j, carry):
            ev = einfo_v[pl.ds(j * 16, 16)]
            srow = ((ev >> 10) * 2 + t) * _HITROWS + ((ev & 1023) >> 2)
            srclist_v[pl.ds(j * 16, 16)] = srow
            return carry

        lax.fori_loop(0, _BPW // 16, srcfill, jnp.int32(0))

        hs = [
            pltpu.async_copy(
                stage_hbm.at[srclist_v.at[pl.ds(q * 128, 128)]],
                (buf_a if q < 2 else buf_b).at[pl.ds((q % 2) * 128, 128)],
                gsem,
            )
            for q in range(_BPW // 128)
        ]
        for h in hs:
            h.wait()

        def assemble(j, carry):
            ev = einfo_v[pl.ds(j * 16, 16)]
            sub = ev & 3
            rows = (j * 16 + iota) & 255
            buf = buf_a
            for c in range(N_EMBED):
                vals = plsc.load_gather(buf, [rows, sub * N_EMBED + c])
                plsc.store_scatter(
                    outT_v, [_splat(c), j * 16 + iota], vals
                )
            return carry

        def assemble_b(j, carry):
            ev = einfo_v[pl.ds(j * 16, 16)]
            sub = ev & 3
            rows = (j * 16 + iota) & 255
            for c in range(N_EMBED):
                vals = plsc.load_gather(buf_b, [rows, sub * N_EMBED + c])
                plsc.store_scatter(
                    outT_v, [_splat(c), j * 16 + iota], vals
                )
            return carry

        lax.fori_loop(0, _BPW // 32, assemble, jnp.int32(0))
        lax.fori_loop(_BPW // 32, _BPW // 16, assemble_b, jnp.int32(0))

        pltpu.async_copy(
            outT_v, out_hbm.at[t, :, pl.ds(bb, _BPW)], osem
        ).wait()


_MESH = plsc.VectorSubcoreMesh(core_axis_name="c", subcore_axis_name="s")
_PARAMS = pltpu.CompilerParams(
    use_tc_tiling_on_sc=True, needs_layout_passes=False
)


@functools.partial(
    pl.kernel,
    mesh=_MESH,
    out_type=jax.ShapeDtypeStruct((_STAGE_ROWS, 128), jnp.float32),
    scratch_types=[
        pltpu.VMEM((BATCH,), jnp.int32),
        pltpu.VMEM((_CAP + 16,), jnp.int32),
        pltpu.VMEM((_SELROWS, 128), jnp.float32),
        pltpu.VMEM((80,), jnp.int32),
        pltpu.VMEM((N_EMBED, _CH), jnp.float32),
        pltpu.VMEM((N_EMBED, _CH), jnp.float32),
        pltpu.VMEM((_HITROWS, 128), jnp.float32),
        pltpu.VMEM((16, 128), jnp.float32),
        pltpu.SemaphoreType.DMA,
    ],
    compiler_params=_PARAMS,
)
def _skipgram_scan(*refs):
    _k1_body(*refs)


@functools.partial(
    pl.kernel,
    mesh=_MESH,
    out_type=jax.ShapeDtypeStruct((2, N_EMBED, BATCH), jnp.float32),
    scratch_types=[
        pltpu.VMEM((_NW // 2 * _SELROWS, 128), jnp.float32),
        pltpu.VMEM((_NW // 2 * _SELROWS, 128), jnp.float32),
        pltpu.VMEM((_BPW,), jnp.int32),
        pltpu.VMEM((_BPW,), jnp.int32),
        pltpu.VMEM((N_EMBED, _BPW), jnp.float32),
        pltpu.SemaphoreType.DMA,
        pltpu.SemaphoreType.DMA,
    ],
    compiler_params=_PARAMS,
)
def _skipgram_assemble(*refs):
    _k2_body(*refs)


def kernel(input_words, output_words, in_embed_weight, out_embed_weight):
    idx0 = input_words.astype(jnp.int32)
    idx1 = output_words.astype(jnp.int32)
    tail0 = in_embed_weight[_TAIL_LO:].reshape(16, 128)
    tail1 = out_embed_weight[_TAIL_LO:].reshape(16, 128)
    stage = _skipgram_scan(idx0, idx1, in_embed_weight.T, out_embed_weight.T,
                           tail0, tail1)
    outT = _skipgram_assemble(stage)
    return jnp.transpose(outT, (0, 2, 1))
